# baseline (speedup 1.0000x reference)
import dataclasses
import functools
import math

import jax
import jax.numpy as jnp
from jax.experimental import pallas as pl
from jax.experimental.pallas import tpu as pltpu
from jax.experimental.pallas import tpu_sc as plsc

_CUTOFF = 5.0
_NUM_BASIS = 16
_LANES = 128
_BLOCK_ROWS = 12800

_SIN_C = (3.1415917330, -5.1676850392, 2.5499267721,
          -5.9839777752e-1, 8.0605215494e-2, -6.0412088560e-3)
_COS_C = (9.9999999989e-01, -4.9348021859e+00, 4.0587118172e+00,
          -1.3352602861e+00, 2.3532082530e-01, -2.5785808394e-02,
          1.9043286625e-03, -8.8690844412e-05)


def _edge_body(a_ref, o_ref):
    a = a_ref[...]
    sq = a * a
    r2 = sq[:, 0:1] + sq[:, 1:2] + sq[:, 2:3]
    inv_r = jax.lax.rsqrt(r2)
    w = (r2 * inv_r) * (1.0 / _CUTOFF)
    q = inv_r * math.sqrt(2.0 / _CUTOFF)
    rows = _BLOCK_ROWS // _LANES
    wp = w.reshape(rows, _LANES)
    qp = q.reshape(rows, _LANES)
    vs = []
    for n in range(1, _NUM_BASIS + 1):
        t = wp * float(n)
        k = jax.lax.round(t * 0.5, jax.lax.RoundingMethod.TO_NEAREST_EVEN)
        m = t - (k + k)
        m2 = m * m
        p = _SIN_C[-1]
        for c in _SIN_C[-2::-1]:
            p = p * m2 + c
        vs.append((p * m) * qp)
    w3 = jnp.stack(vs, axis=-1)
    o_ref[...] = w3.reshape(_BLOCK_ROWS, _NUM_BASIS)


def _edge_call(edge_attr):
    rows = edge_attr.shape[0]
    grid = rows // _BLOCK_ROWS
    return pl.pallas_call(
        _edge_body,
        grid=(grid,),
        in_specs=[pl.BlockSpec((_BLOCK_ROWS, 3), lambda i: (i, 0))],
        out_specs=pl.BlockSpec((_BLOCK_ROWS, _NUM_BASIS), lambda i: (i, 0)),
        out_shape=jax.ShapeDtypeStruct((rows, _NUM_BASIS), jnp.float32),
    )(edge_attr)


def _node_gather(x_idx, W_x, W_z):
    n_real = x_idx.shape[0]
    n_workers = 32
    b_per_w = 3128
    pairs = b_per_w // 2
    last_base = n_real - b_per_w
    table = jnp.concatenate([W_x, W_z], axis=1)
    mesh = plsc.VectorSubcoreMesh(core_axis_name="c", subcore_axis_name="s")

    cp = pltpu.CompilerParams()
    if "needs_layout_passes" in pltpu.CompilerParams.__dataclass_fields__:
        cp = dataclasses.replace(cp, needs_layout_passes=False)

    @functools.partial(
        pl.kernel, mesh=mesh, compiler_params=cp,
        out_type=[jax.ShapeDtypeStruct((n_real * 8,), jnp.float32),
                  jax.ShapeDtypeStruct((n_real * 8,), jnp.float32)],
        scratch_types=[pltpu.VMEM((b_per_w,), jnp.int32),
                       pltpu.VMEM((pairs * 16,), jnp.float32),
                       pltpu.VMEM((pairs * 16,), jnp.float32),
                       pltpu.VMEM((100, 16), jnp.float32)])
    def knl(table_hbm, idx_hbm, outx_hbm, outz_hbm, idx_v, rx_v, rz_v, tab_v):
        wid = jax.lax.axis_index("s") * 2 + jax.lax.axis_index("c")
        base = jnp.where(wid == n_workers - 1, last_base, wid * b_per_w)
        pltpu.sync_copy(table_hbm, tab_v)
        pltpu.sync_copy(idx_hbm.at[pl.ds(base, b_per_w)], idx_v)
        lane = jax.lax.iota(jnp.int32, 16)
        pat = jnp.where(lane < 8, 0, 1)
        colx = jnp.where(lane < 8, lane, lane - 8)
        colz = colx + 8

        @pl.loop(0, pairs)
        def _(kk):
            iv = plsc.load_gather(idx_v, [pat + (kk + kk)])
            vx = plsc.load_gather(tab_v, [iv, colx])
            vz = plsc.load_gather(tab_v, [iv, colz])
            rx_v[pl.ds(kk * 16, 16)] = vx
            rz_v[pl.ds(kk * 16, 16)] = vz

        pltpu.sync_copy(rx_v, outx_hbm.at[pl.ds(base * 8, b_per_w * 8)])
        pltpu.sync_copy(rz_v, outz_hbm.at[pl.ds(base * 8, b_per_w * 8)])

    outx, outz = knl(table, x_idx)
    return outx.reshape(n_real, 8), outz.reshape(n_real, 8)


def kernel(x, edge_attr, W_x, W_z):
    h_edge = _edge_call(edge_attr)
    h_node_x, h_node_z = _node_gather(x, W_x, W_z)
    return (h_node_x, h_node_z, h_edge)


# speedup vs baseline: 1.7927x; 1.7927x over previous
import dataclasses
import functools
import math

import jax
import jax.numpy as jnp
from jax.experimental import pallas as pl
from jax.experimental.pallas import tpu as pltpu
from jax.experimental.pallas import tpu_sc as plsc

_CUTOFF = 5.0
_NUM_BASIS = 16
_LANES = 128
_BLOCK_ROWS = 2560

_SIN_C = (3.1415917330, -5.1676850392, 2.5499267721,
          -5.9839777752e-1, 8.0605215494e-2, -6.0412088560e-3)
_COS_C = (9.9999999989e-01, -4.9348021859e+00, 4.0587118172e+00,
          -1.3352602861e+00, 2.3532082530e-01, -2.5785808394e-02,
          1.9043286625e-03, -8.8690844412e-05)


def _edge_body(a_ref, o_ref):
    a = a_ref[...]
    sq = a * a
    r2 = sq[:, 0:1] + sq[:, 1:2] + sq[:, 2:3]
    inv_r = jax.lax.rsqrt(r2)
    u = (r2 * inv_r) * (0.5 / _CUTOFF)
    q = inv_r * math.sqrt(2.0 / _CUTOFF)
    rows = _BLOCK_ROWS // _LANES
    up = u.reshape(rows, _LANES)
    qp = q.reshape(rows, _LANES)
    k = jax.lax.round(up, jax.lax.RoundingMethod.TO_NEAREST_EVEN)
    t = (up - k) * 2.0
    t2 = t * t
    s = _SIN_C[-1]
    for c in _SIN_C[-2::-1]:
        s = s * t2 + c
    s1 = s * t
    cp = _COS_C[-1]
    for c in _COS_C[-2::-1]:
        cp = cp * t2 + c
    twoc = cp + cp
    vs = [s1 * qp]
    s_prev, s_cur = s1, twoc * s1
    vs.append(s_cur * qp)
    for _ in range(_NUM_BASIS - 2):
        s_prev, s_cur = s_cur, twoc * s_cur - s_prev
        vs.append(s_cur * qp)
    w3 = jnp.stack(vs, axis=-1)
    o_ref[...] = w3.reshape(_BLOCK_ROWS, _NUM_BASIS)


def _edge_call(edge_attr):
    rows = edge_attr.shape[0]
    grid = rows // _BLOCK_ROWS
    return pl.pallas_call(
        _edge_body,
        grid=(grid,),
        in_specs=[pl.BlockSpec((_BLOCK_ROWS, 3), lambda i: (i, 0))],
        out_specs=pl.BlockSpec((_BLOCK_ROWS, _NUM_BASIS), lambda i: (i, 0)),
        out_shape=jax.ShapeDtypeStruct((rows, _NUM_BASIS), jnp.float32),
    )(edge_attr)


def _node_gather(x_idx, W_x, W_z):
    n_real = x_idx.shape[0]
    n_workers = 32
    b_per_w = 3200
    B = n_workers * b_per_w
    idx = jnp.zeros((B,), x_idx.dtype).at[:n_real].set(x_idx)
    table = jnp.concatenate([W_x, W_z], axis=1)
    mesh = plsc.VectorSubcoreMesh(core_axis_name="c", subcore_axis_name="s")

    cp = pltpu.CompilerParams()
    if "needs_layout_passes" in pltpu.CompilerParams.__dataclass_fields__:
        cp = dataclasses.replace(cp, needs_layout_passes=False)

    @functools.partial(
        pl.kernel, mesh=mesh, compiler_params=cp,
        out_type=jax.ShapeDtypeStruct((B * 16,), jnp.float32),
        scratch_types=[pltpu.VMEM((b_per_w,), jnp.int32),
                       pltpu.VMEM((b_per_w * 16,), jnp.float32),
                       pltpu.VMEM((100, 16), jnp.float32)])
    def knl(table_hbm, idx_hbm, out_hbm, idx_v, rows_v, tab_v):
        wid = jax.lax.axis_index("s") * 2 + jax.lax.axis_index("c")
        base = wid * b_per_w
        pltpu.sync_copy(table_hbm, tab_v)
        pltpu.sync_copy(idx_hbm.at[pl.ds(base, b_per_w)], idx_v)
        cols = jax.lax.iota(jnp.int32, 16)

        @pl.loop(0, b_per_w)
        def _(k):
            iv = plsc.load_gather(idx_v, [jnp.full((16,), k, jnp.int32)])
            vals = plsc.load_gather(tab_v, [iv, cols])
            rows_v[pl.ds(k * 16, 16)] = vals

        pltpu.sync_copy(rows_v, out_hbm.at[pl.ds(base * 16, b_per_w * 16)])

    out = knl(table, idx).reshape(B, 16)
    return out[:n_real, :8], out[:n_real, 8:]


def kernel(x, edge_attr, W_x, W_z):
    h_edge = _edge_call(edge_attr)
    h_node_x, h_node_z = _node_gather(x, W_x, W_z)
    return (h_node_x, h_node_z, h_edge)
